# baseline (speedup 1.0000x reference)
import jax
import jax.numpy as jnp
from jax import lax
from jax.experimental import pallas as pl
from jax.experimental.pallas import tpu as pltpu
from jax.experimental.pallas import tpu_sc as plsc

_NUM_CORES = 2
_NUM_SUBCORES = 16
_NUM_WORKERS = _NUM_CORES * _NUM_SUBCORES
_CHUNK = 256


def kernel(ids, table):
    batch, hist = ids.shape
    vocab, d = table.shape
    num_indices = batch * hist
    per_worker = num_indices // _NUM_WORKERS

    flat = ids.reshape(num_indices)
    q = flat >> 1
    table2 = table.reshape(vocab // 2, 2 * d)

    mesh = plsc.VectorSubcoreMesh(core_axis_name="c", subcore_axis_name="s")

    @pl.kernel(
        out_type=jax.ShapeDtypeStruct((num_indices, 2 * d), table.dtype),
        mesh=mesh,
        scratch_types=[
            pltpu.VMEM((_CHUNK,), jnp.int32),
            pltpu.VMEM((_CHUNK, 2 * d), table.dtype),
            pltpu.SemaphoreType.DMA,
        ],
    )
    def gather_kernel(table_hbm, ids_hbm, out_hbm, idx_v, rows_v, sem):
        wid = lax.axis_index("s") * _NUM_CORES + lax.axis_index("c")
        base = wid * per_worker

        @pl.loop(0, per_worker, step=_CHUNK)
        def _(off):
            pltpu.sync_copy(ids_hbm.at[pl.ds(base + off, _CHUNK)], idx_v)
            pltpu.async_copy(table_hbm.at[idx_v], rows_v, sem).wait()
            pltpu.sync_copy(rows_v, out_hbm.at[pl.ds(base + off, _CHUNK)])

    out2 = gather_kernel(table2, q)
    r = (flat & 1)[:, None]
    out = jnp.where(r == 0, out2[:, :d], out2[:, d:])
    return out.reshape(batch, hist, d)


# speedup vs baseline: 1.2636x; 1.2636x over previous
import dataclasses

import jax
import jax.numpy as jnp
from jax import lax
from jax.experimental import pallas as pl
from jax.experimental.pallas import tpu as pltpu
from jax.experimental.pallas import tpu_sc as plsc

_NUM_CORES = 2
_NUM_SUBCORES = 16
_NUM_WORKERS = _NUM_CORES * _NUM_SUBCORES
_CHUNK = 400


def kernel(ids, table):
    batch, hist = ids.shape
    vocab, d = table.shape
    num_indices = batch * hist
    per_worker = num_indices // _NUM_WORKERS
    flat = ids.reshape(num_indices)

    mesh = plsc.VectorSubcoreMesh(core_axis_name="c", subcore_axis_name="s")
    cp = dataclasses.replace(pltpu.CompilerParams(), use_tc_tiling_on_sc=False)

    @pl.kernel(
        out_type=jax.ShapeDtypeStruct((num_indices, d), table.dtype),
        mesh=mesh,
        scratch_types=[
            pltpu.VMEM((_CHUNK,), jnp.int32),
            pltpu.VMEM((_CHUNK, d), table.dtype),
            pltpu.SemaphoreType.DMA,
        ],
        compiler_params=cp,
    )
    def gather_kernel(table_hbm, ids_hbm, out_hbm, idx_v, rows_v, sem):
        wid = lax.axis_index("s") * _NUM_CORES + lax.axis_index("c")
        base = wid * per_worker

        @pl.loop(0, per_worker, step=_CHUNK)
        def _(off):
            pltpu.sync_copy(ids_hbm.at[pl.ds(base + off, _CHUNK)], idx_v)
            pltpu.async_copy(table_hbm.at[idx_v], rows_v, sem).wait()
            pltpu.sync_copy(rows_v, out_hbm.at[pl.ds(base + off, _CHUNK)])

    out = gather_kernel(table, flat)
    return out.reshape(batch, hist, d)
